# baseline (speedup 1.0000x reference)
import functools

import jax
import jax.numpy as jnp
from jax import lax
from jax.experimental import pallas as pl
from jax.experimental.pallas import tpu as pltpu
from jax.experimental.pallas import tpu_sc as plsc

N_NODES = 10000
D_IN = 128
D_HID = 128
N_CLASSES = 3
N_EDGES = 320000

N_PAD = 10240
N_ACC = 10112
E_REAL = N_EDGES + N_NODES
NW = 32
G = 32
NGRP = 326
EPW = NGRP * G
E_PAD = NW * EPW
SDPAD = 4
BR = 2560
RPT = N_ACC // 16


def _tc_pre_body(x_ref, w_ref, a2_ref, h_ref, ap_ref):
    h = jnp.dot(x_ref[...], w_ref[...], preferred_element_type=jnp.float32)
    h_ref[...] = h
    ap_ref[...] = jnp.dot(h, a2_ref[...], preferred_element_type=jnp.float32)


def _tc_pre(x_pad, W1, a2):
    br = BR
    return pl.pallas_call(
        _tc_pre_body,
        grid=(N_PAD // br,),
        in_specs=[
            pl.BlockSpec((br, D_IN), lambda i: (i, 0)),
            pl.BlockSpec((D_IN, D_HID), lambda i: (0, 0)),
            pl.BlockSpec((D_HID, 2), lambda i: (0, 0)),
        ],
        out_specs=[
            pl.BlockSpec((br, D_HID), lambda i: (i, 0)),
            pl.BlockSpec((br, 2), lambda i: (i, 0)),
        ],
        out_shape=[
            jax.ShapeDtypeStruct((N_PAD, D_HID), jnp.float32),
            jax.ShapeDtypeStruct((N_PAD, 2), jnp.float32),
        ],
    )(x_pad, W1, a2)


def _sc1_body(sd_hbm, h_hbm, ap_hbm,
              msg_hbm, den_hbm,
              ap_v, rows_v, sd_v, w_v, den_v, acc_sh, sems):
    c = lax.axis_index("c")
    s = lax.axis_index("s")
    wid = c * 16 + s
    gg0 = wid * NGRP
    lane = jnp.arange(16, dtype=jnp.int32)
    gsem = sems[0:3]
    ssem = sems[3:6]
    isem = sems[6:12]

    pltpu.sync_copy(ap_hbm.at[pl.ds(0, N_ACC * 2)], ap_v)

    zf = jnp.zeros((16,), jnp.float32)

    def zden(i, _):
        den_v[pl.ds(i * 16, 16)] = zf
        return 0
    lax.fori_loop(0, N_PAD // 16, zden, 0)

    def zrow(i, _):
        for k in range(8):
            rows_v[0, i, pl.ds(k * 16, 16)] = zf
        return 0
    lax.fori_loop(0, G, zrow, 0)

    for k in range(RPT // G):
        pltpu.sync_copy(rows_v.at[0], acc_sh.at[pl.ds(s * RPT + k * G, G)])
    pltpu.sync_copy(rows_v.at[0].at[pl.ds(0, RPT % G)],
                    acc_sh.at[pl.ds(s * RPT + (RPT // G) * G, RPT % G)])
    plsc.subcore_barrier()

    def idx_start(gg, b):
        pltpu.async_copy(sd_hbm.at[gg], sd_v.at[b], isem[b])

    def idx_wait(b):
        pltpu.make_async_copy(sd_hbm.at[0], sd_v.at[b], isem[b]).wait()

    def gather_start(b, r):
        pltpu.async_copy(h_hbm.at[sd_v.at[b].at[0]], rows_v.at[r], gsem[r])

    def gather_wait(r):
        pltpu.make_async_copy(h_hbm.at[sd_v.at[0].at[0]], rows_v.at[r],
                              gsem[r]).wait()

    def scatter_start(b, r):
        pltpu.async_copy(rows_v.at[r], acc_sh.at[sd_v.at[b].at[1]], ssem[r],
                         add=True)

    def scatter_wait(r):
        pltpu.make_async_copy(rows_v.at[r], acc_sh.at[sd_v.at[0].at[1]],
                              ssem[r]).wait()

    def w_compute(b, gg):
        for j in range(G // 16):
            si = sd_v[b, 0, pl.ds(j * 16, 16)]
            di = sd_v[b, 1, pl.ds(j * 16, 16)]
            a = (plsc.load_gather(ap_v, [si * 2])
                 + plsc.load_gather(ap_v, [di * 2 + 1]))
            a = jnp.maximum(a, 0.2 * a)
            eid = gg * G + j * 16 + lane
            w = jnp.where(eid < E_REAL, jnp.exp(a), 0.0)
            w_v[pl.ds(j * 16, 16)] = w
            plsc.addupdate_scatter(den_v, [di], w)

    RU = 8

    def scale(r):
        def body(i, _):
            j0 = i * RU
            spl = [plsc.load_gather(w_v, [jnp.full((16,), j0 + q, jnp.int32)])
                   for q in range(RU)]
            for q in range(RU):
                for k in range(8):
                    rows_v[r, j0 + q, pl.ds(k * 16, 16)] = (
                        rows_v[r, j0 + q, pl.ds(k * 16, 16)] * spl[q])
            return 0
        lax.fori_loop(0, G // RU, body, 0)

    def compute_and_scatter(b, r, gg):
        gather_wait(r)
        w_compute(b, gg)
        scale(r)
        scatter_start(b, r)

    for b in range(4):
        idx_start(gg0 + b, b)
    idx_wait(0)
    gather_start(0, 0)

    idx_wait(1)
    gather_start(1, 1)
    idx_start(gg0 + 4, 4)
    compute_and_scatter(0, 0, gg0)

    idx_wait(2)
    gather_start(2, 2)
    idx_start(gg0 + 5, 5)
    compute_and_scatter(1, 1, gg0 + 1)

    def six(k, _):
        g = 2 + 6 * k
        for u in range(6):
            b = (2 + u) % 6
            r = (2 + u) % 3
            scatter_wait((r + 1) % 3)
            idx_wait((b + 1) % 6)
            gather_start((b + 1) % 6, (r + 1) % 3)
            idx_start(gg0 + g + u + 4, (b + 4) % 6)
            compute_and_scatter(b, r, gg0 + g + u)
        return 0
    lax.fori_loop(0, (NGRP - 2) // 6, six, 0)

    scatter_wait(0)
    scatter_wait(1)
    gather_wait(2)
    idx_wait(3)
    idx_wait(4)
    idx_wait(5)

    plsc.subcore_barrier()
    pltpu.sync_copy(acc_sh.at[pl.ds(s * RPT, RPT)],
                    msg_hbm.at[c].at[pl.ds(s * RPT, RPT)])
    pltpu.sync_copy(den_v, den_hbm.at[wid])


@functools.partial(
    pl.kernel,
    out_type=(
        jax.ShapeDtypeStruct((2, N_PAD, D_HID), jnp.float32),
        jax.ShapeDtypeStruct((NW, N_PAD), jnp.float32),
    ),
    mesh=plsc.VectorSubcoreMesh(core_axis_name="c", subcore_axis_name="s"),
    compiler_params=pltpu.CompilerParams(needs_layout_passes=False),
    scratch_types=(
        pltpu.VMEM((N_ACC * 2,), jnp.float32),
        pltpu.VMEM((3, G, D_HID), jnp.float32),
        pltpu.VMEM((6, 2, G), jnp.int32),
        pltpu.VMEM((G,), jnp.float32),
        pltpu.VMEM((N_PAD,), jnp.float32),
        pltpu.VMEM_SHARED((N_ACC, D_HID), jnp.float32),
    ) + (pltpu.SemaphoreType.DMA,) * 12,
)
def _sc_layer1(sd_hbm, h_hbm, ap_hbm, msg_hbm, den_hbm,
               ap_v, rows_v, sd_v, w_v, den_v, acc_sh, *sems):
    _sc1_body(sd_hbm, h_hbm, ap_hbm, msg_hbm, den_hbm,
              ap_v, rows_v, sd_v, w_v, den_v, acc_sh, sems)


def _tc_mid_body(msg_ref, den_ref, b1_ref, w2_ref, a22_ref, hp2_ref):
    p = msg_ref[0] + msg_ref[1]
    d = jnp.sum(den_ref[...], axis=0)
    o = p / (d[:, None] + 1e-16) + b1_ref[...]
    o = jnp.maximum(o, 0.0)
    h2 = jnp.dot(o, w2_ref[...], preferred_element_type=jnp.float32)
    a22 = a22_ref[...]
    as2 = jnp.sum(h2 * a22[0:1, :], axis=1)
    ad2 = jnp.sum(h2 * a22[1:2, :], axis=1)
    hp2_ref[...] = jnp.concatenate([h2, as2[:, None], ad2[:, None]], axis=1)


def _tc_mid(msg1, den1, b1_2d, W2, a22):
    br = BR
    return pl.pallas_call(
        _tc_mid_body,
        grid=(N_PAD // br,),
        in_specs=[
            pl.BlockSpec((2, br, D_HID), lambda i: (0, i, 0)),
            pl.BlockSpec((NW, br), lambda i: (0, i)),
            pl.BlockSpec((1, D_HID), lambda i: (0, 0)),
            pl.BlockSpec((D_HID, N_CLASSES), lambda i: (0, 0)),
            pl.BlockSpec((2, N_CLASSES), lambda i: (0, 0)),
        ],
        out_specs=pl.BlockSpec((br, 5), lambda i: (i, 0)),
        out_shape=jax.ShapeDtypeStruct((N_PAD, 5), jnp.float32),
    )(msg1, den1, b1_2d, W2, a22)


def _sc2_body(src_hbm, dst_hbm, hp2_hbm, msg_hbm, den_hbm,
              hp2_v, src_v, dst_v, acc_v, den_v):
    c = lax.axis_index("c")
    s = lax.axis_index("s")
    wid = c * 16 + s
    base = wid * EPW

    pltpu.sync_copy(hp2_hbm, hp2_v)
    pltpu.sync_copy(src_hbm.at[pl.ds(base, EPW)], src_v)
    pltpu.sync_copy(dst_hbm.at[pl.ds(base, EPW)], dst_v)

    zf = jnp.zeros((16,), jnp.float32)

    def zden(i, _):
        den_v[pl.ds(i * 16, 16)] = zf
        return 0
    lax.fori_loop(0, N_PAD // 16, zden, 0)

    def zacc(i, _):
        acc_v[pl.ds(i * 16, 16)] = zf
        return 0
    lax.fori_loop(0, N_PAD * 4 // 16, zacc, 0)

    lane = jnp.arange(16, dtype=jnp.int32)

    def step(i, _):
        e0 = i * 16
        si = src_v[pl.ds(e0, 16)]
        di = dst_v[pl.ds(e0, 16)]
        si5 = si * 5
        a = (plsc.load_gather(hp2_v, [si5 + 3])
             + plsc.load_gather(hp2_v, [di * 5 + 4]))
        a = jnp.maximum(a, 0.2 * a)
        eid = base + e0 + lane
        w = jnp.where(eid < E_REAL, jnp.exp(a), 0.0)
        plsc.addupdate_scatter(den_v, [di], w)
        di4 = di * 4
        for col in range(N_CLASSES):
            hv = plsc.load_gather(hp2_v, [si5 + col])
            plsc.addupdate_scatter(acc_v, [di4 + col], w * hv)
        return 0

    lax.fori_loop(0, EPW // 16, step, 0)

    pltpu.sync_copy(acc_v, msg_hbm.at[wid])
    pltpu.sync_copy(den_v, den_hbm.at[wid])


@functools.partial(
    pl.kernel,
    out_type=(
        jax.ShapeDtypeStruct((NW, N_PAD * 4), jnp.float32),
        jax.ShapeDtypeStruct((NW, N_PAD), jnp.float32),
    ),
    mesh=plsc.VectorSubcoreMesh(core_axis_name="c", subcore_axis_name="s"),
    compiler_params=pltpu.CompilerParams(needs_layout_passes=False),
    scratch_types=(
        pltpu.VMEM((N_PAD * 5,), jnp.float32),
        pltpu.VMEM((EPW,), jnp.int32),
        pltpu.VMEM((EPW,), jnp.int32),
        pltpu.VMEM((N_PAD * 4,), jnp.float32),
        pltpu.VMEM((N_PAD,), jnp.float32),
    ),
)
def _sc_layer2(src_hbm, dst_hbm, hp2_hbm, msg_hbm, den_hbm,
               hp2_v, src_v, dst_v, acc_v, den_v):
    _sc2_body(src_hbm, dst_hbm, hp2_hbm, msg_hbm, den_hbm,
              hp2_v, src_v, dst_v, acc_v, den_v)


def _tc_final_body(msg_ref, den_ref, b2_ref, out_ref):
    sm = jnp.sum(msg_ref[...], axis=0)
    d = jnp.sum(den_ref[...], axis=0)
    out_ref[...] = sm / (d[:, None] + 1e-16) + b2_ref[...]


def _tc_final(msg2, den2, b2p):
    br = 1024
    return pl.pallas_call(
        _tc_final_body,
        grid=(N_PAD // br,),
        in_specs=[
            pl.BlockSpec((NW, br, 4), lambda i: (0, i, 0)),
            pl.BlockSpec((NW, br), lambda i: (0, i)),
            pl.BlockSpec((1, 4), lambda i: (0, 0)),
        ],
        out_specs=pl.BlockSpec((br, 4), lambda i: (i, 0)),
        out_shape=jax.ShapeDtypeStruct((N_PAD, 4), jnp.float32),
    )(msg2, den2, b2p)


def kernel(x, edge_index, W1, a_src1, a_dst1, b1, W2, a_src2, a_dst2, b2):
    loop = jnp.arange(N_NODES, dtype=edge_index.dtype)
    src = jnp.concatenate([edge_index[0], loop])
    dst = jnp.concatenate([edge_index[1], loop])
    src_p = jnp.zeros((E_PAD,), jnp.int32).at[:E_REAL].set(src.astype(jnp.int32))
    dst_p = jnp.zeros((E_PAD,), jnp.int32).at[:E_REAL].set(dst.astype(jnp.int32))
    sd = jnp.stack([src_p.reshape(NW * NGRP, G), dst_p.reshape(NW * NGRP, G)],
                   axis=1)
    sd = jnp.concatenate([sd, jnp.zeros((SDPAD, 2, G), jnp.int32)], axis=0)

    x_pad = jnp.zeros((N_PAD, D_IN), jnp.float32).at[:N_NODES].set(x)
    a2 = jnp.stack([a_src1, a_dst1], axis=1)
    a22 = jnp.stack([a_src2, a_dst2], axis=0)
    b1_2d = b1[None, :]
    b2p = jnp.concatenate([b2, jnp.zeros((1,), jnp.float32)])[None, :]

    h, ap = _tc_pre(x_pad, W1, a2)
    msg1, den1 = _sc_layer1(sd, h, ap.reshape(N_PAD * 2))
    hp2 = _tc_mid(msg1, den1, b1_2d, W2, a22)
    msg2, den2 = _sc_layer2(src_p, dst_p, hp2.reshape(N_PAD * 5))
    out = _tc_final(msg2.reshape(NW, N_PAD, 4), den2, b2p)
    return out[:N_NODES, :N_CLASSES]
